# baseline (speedup 1.0000x reference)
import jax
import jax.numpy as jnp
from jax import lax
from jax.experimental import pallas as pl
from jax.experimental.pallas import tpu as pltpu
from jax.experimental.pallas import tpu_sc as plsc

BATCH = 16384
FIELDS = 26
DIM = 16
D_IN = FIELDS * DIM
D_PAD = 512
NUM_SLOTS = BATCH * D_PAD // DIM
T_ROWS = BATCH * D_PAD // 128

W = 128
IDX_ROWS = NUM_SLOTS // W
N_WORKERS = 32
ROWS_PER_WORKER = IDX_ROWS // N_WORKERS

BB = 2048


def _sc_gather_tiled(table, idx2d):
    mesh = plsc.VectorSubcoreMesh(core_axis_name="core", subcore_axis_name="subcore")

    @pl.kernel(
        out_type=jax.ShapeDtypeStruct((NUM_SLOTS, DIM), table.dtype),
        mesh=mesh,
        scratch_types=[
            pltpu.VMEM((W,), jnp.int32),
            pltpu.VMEM((W, DIM), jnp.float32),
        ],
        compiler_params=pltpu.CompilerParams(use_tc_tiling_on_sc=False),
    )
    def gather_kernel(tab_hbm, idx_hbm, out_hbm, idx_v, rows_v):
        wid = lax.axis_index("subcore") * 2 + lax.axis_index("core")

        @pl.loop(0, ROWS_PER_WORKER)
        def _(w):
            row = wid * ROWS_PER_WORKER + w
            pltpu.sync_copy(idx_hbm.at[row], idx_v)
            pltpu.sync_copy(tab_hbm.at[idx_v], rows_v)
            pltpu.sync_copy(rows_v, out_hbm.at[pl.ds(row * W, W)])

    return gather_kernel(table, idx2d)


def _mlp_block(t_ref, w1, b1, w2, b2, w3, b3, w4, b4, w5, b5, out_ref):
    t = t_ref[...].reshape(BB // 8, 4, 8, 128)
    h = jnp.concatenate(
        [t[:, j].reshape(BB, 128) for j in range(4)], axis=1
    )
    h = jnp.maximum(jnp.dot(h, w1[...], preferred_element_type=jnp.float32) + b1[...], 0.0)
    h = jnp.maximum(jnp.dot(h, w2[...], preferred_element_type=jnp.float32) + b2[...], 0.0)
    h = jnp.maximum(jnp.dot(h, w3[...], preferred_element_type=jnp.float32) + b3[...], 0.0)
    h = jnp.maximum(jnp.dot(h, w4[...], preferred_element_type=jnp.float32) + b4[...], 0.0)
    o = jnp.dot(h, w5[...], preferred_element_type=jnp.float32) + b5[...]
    out_ref[...] = jax.nn.sigmoid(o)


def _mlp(tarr, W1p, b1, W2, b2, W3, b3, W4, b4, W5, b5):
    full = lambda a: pl.BlockSpec(a.shape, lambda i: (0,) * a.ndim)
    return pl.pallas_call(
        _mlp_block,
        grid=(BATCH // BB,),
        in_specs=[
            pl.BlockSpec((BB * 4, 128), lambda i: (i, 0)),
            full(W1p), full(b1), full(W2), full(b2), full(W3), full(b3),
            full(W4), full(b4), full(W5), full(b5),
        ],
        out_specs=pl.BlockSpec((BB, 1), lambda i: (i, 0)),
        out_shape=jax.ShapeDtypeStruct((BATCH, 1), jnp.float32),
    )(tarr, W1p, b1, W2, b2, W3, b3, W4, b4, W5, b5)


def kernel(x, table, W1, b1, W2, b2, W3, b3, W4, b4, W5, b5):
    xpad = jnp.pad(x, ((0, 0), (0, 32 - FIELDS)))
    perm_idx = (
        xpad.reshape(BATCH // 8, 8, 4, 8)
        .transpose(0, 2, 1, 3)
        .reshape(IDX_ROWS, W)
    )
    tarr = _sc_gather_tiled(table, perm_idx).reshape(T_ROWS, 128)
    W1p = jnp.zeros((D_PAD, 512), jnp.float32).at[:D_IN].set(W1)
    return _mlp(
        tarr,
        W1p, b1.reshape(1, -1),
        W2, b2.reshape(1, -1),
        W3, b3.reshape(1, -1),
        W4, b4.reshape(1, -1),
        W5, b5.reshape(1, -1),
    )


# speedup vs baseline: 2.1927x; 2.1927x over previous
import jax
import jax.numpy as jnp
from jax.experimental import pallas as pl
from jax.experimental.pallas import tpu as pltpu
from jax.experimental.pallas import tpu_sc as plsc

BATCH = 16384
FIELDS = 26
DIM = 16
VOCAB = 1000000
NUM_IDX = BATCH * FIELDS

GATHER_WINDOW = 128
IDX_ROWS = NUM_IDX // GATHER_WINDOW

RELAY_BN = 65536

BB = 2048


def _relay_block(tt_ref, out_ref):
    tT = tt_ref[...].T
    s = RELAY_BN // 8
    out_ref[...] = jnp.concatenate(
        [tT[c * s:(c + 1) * s, :] for c in range(8)], axis=1
    )


def _relayout_table(table):
    tt = table.T
    return pl.pallas_call(
        _relay_block,
        grid=(16,),
        in_specs=[pl.BlockSpec((DIM, RELAY_BN), lambda j: (0, j))],
        out_specs=pl.BlockSpec((RELAY_BN // 8, 128), lambda j: (j, 0)),
        out_shape=jax.ShapeDtypeStruct((16 * RELAY_BN // 8, 128), jnp.float32),
    )(tt)


def _sc_gather(table_lin, idx2d):
    mesh = plsc.VectorSubcoreMesh(core_axis_name="core", subcore_axis_name="subcore")

    @pl.kernel(
        out_type=jax.ShapeDtypeStruct((NUM_IDX, DIM), table_lin.dtype),
        mesh=mesh,
        compiler_params=pltpu.CompilerParams(use_tc_tiling_on_sc=False),
    )
    def gather_kernel(tab_hbm, idx_hbm, out_hbm):
        def body(idx_vmem, out_vmem):
            pltpu.sync_copy(tab_hbm.at[idx_vmem.at[0]], out_vmem)

        pltpu.emit_pipeline(
            body,
            grid=(IDX_ROWS,),
            in_specs=[pl.BlockSpec((1, GATHER_WINDOW), index_map=lambda i: (i, 0))],
            out_specs=[pl.BlockSpec((GATHER_WINDOW, DIM), index_map=lambda i: (i, 0))],
            core_axis_name=("core", "subcore"),
            dimension_semantics=(pltpu.PARALLEL,),
        )(idx_hbm, out_hbm)

    return gather_kernel(table_lin, idx2d)


def _mlp_block(emb_ref, w1, b1, w2, b2, w3, b3, w4, b4, w5, b5, out_ref):
    h = emb_ref[...]
    h = jnp.maximum(jnp.dot(h, w1[...], preferred_element_type=jnp.float32) + b1[...], 0.0)
    h = jnp.maximum(jnp.dot(h, w2[...], preferred_element_type=jnp.float32) + b2[...], 0.0)
    h = jnp.maximum(jnp.dot(h, w3[...], preferred_element_type=jnp.float32) + b3[...], 0.0)
    h = jnp.maximum(jnp.dot(h, w4[...], preferred_element_type=jnp.float32) + b4[...], 0.0)
    o = jnp.dot(h, w5[...], preferred_element_type=jnp.float32) + b5[...]
    out_ref[...] = jax.nn.sigmoid(o)


def _mlp(emb, W1, b1, W2, b2, W3, b3, W4, b4, W5, b5):
    full = lambda a: pl.BlockSpec(a.shape, lambda i: (0,) * a.ndim)
    return pl.pallas_call(
        _mlp_block,
        grid=(BATCH // BB,),
        in_specs=[
            pl.BlockSpec((BB, FIELDS * DIM), lambda i: (i, 0)),
            full(W1), full(b1), full(W2), full(b2), full(W3), full(b3),
            full(W4), full(b4), full(W5), full(b5),
        ],
        out_specs=pl.BlockSpec((BB, 1), lambda i: (i, 0)),
        out_shape=jax.ShapeDtypeStruct((BATCH, 1), jnp.float32),
    )(emb, W1, b1, W2, b2, W3, b3, W4, b4, W5, b5)


def kernel(x, table, W1, b1, W2, b2, W3, b3, W4, b4, W5, b5):
    table_lin = _relayout_table(table).reshape(16 * RELAY_BN, DIM)
    xs = (x & ~0xFFFF) | ((x & 0x1FFF) << 3) | ((x >> 13) & 7)
    idx2d = xs.reshape(IDX_ROWS, GATHER_WINDOW)
    rows = _sc_gather(table_lin, idx2d)
    emb = rows.reshape(BATCH, FIELDS * DIM)
    return _mlp(
        emb,
        W1, b1.reshape(1, -1),
        W2, b2.reshape(1, -1),
        W3, b3.reshape(1, -1),
        W4, b4.reshape(1, -1),
        W5, b5.reshape(1, -1),
    )


# speedup vs baseline: 2.2234x; 1.0140x over previous
import jax
import jax.numpy as jnp
from jax.experimental import pallas as pl
from jax.experimental.pallas import tpu as pltpu
from jax.experimental.pallas import tpu_sc as plsc

BATCH = 16384
FIELDS = 26
DIM = 16
VOCAB = 1000000
NUM_IDX = BATCH * FIELDS

GATHER_WINDOW = 128
IDX_ROWS = NUM_IDX // GATHER_WINDOW

RELAY_BN = 65536

BB = 2048


def _relay_block(tt_ref, out_ref):
    s = RELAY_BN // 8
    for c in range(8):
        out_ref[:, 16 * c:16 * (c + 1)] = tt_ref[:, c * s:(c + 1) * s].T


def _relayout_table(table):
    tt = table.T
    return pl.pallas_call(
        _relay_block,
        grid=(16,),
        in_specs=[pl.BlockSpec((DIM, RELAY_BN), lambda j: (0, j))],
        out_specs=pl.BlockSpec((RELAY_BN // 8, 128), lambda j: (j, 0)),
        out_shape=jax.ShapeDtypeStruct((16 * RELAY_BN // 8, 128), jnp.float32),
    )(tt)


def _sc_gather(table_lin, idx2d):
    mesh = plsc.VectorSubcoreMesh(core_axis_name="core", subcore_axis_name="subcore")

    @pl.kernel(
        out_type=jax.ShapeDtypeStruct((NUM_IDX, DIM), table_lin.dtype),
        mesh=mesh,
        compiler_params=pltpu.CompilerParams(use_tc_tiling_on_sc=False),
    )
    def gather_kernel(tab_hbm, idx_hbm, out_hbm):
        def body(idx_vmem, out_vmem):
            pltpu.sync_copy(tab_hbm.at[idx_vmem.at[0]], out_vmem)

        pltpu.emit_pipeline(
            body,
            grid=(IDX_ROWS,),
            in_specs=[pl.BlockSpec((1, GATHER_WINDOW), index_map=lambda i: (i, 0))],
            out_specs=[pl.BlockSpec((GATHER_WINDOW, DIM), index_map=lambda i: (i, 0))],
            core_axis_name=("core", "subcore"),
            dimension_semantics=(pltpu.PARALLEL,),
        )(idx_hbm, out_hbm)

    return gather_kernel(table_lin, idx2d)


def _mlp_block(emb_ref, w1, b1, w2, b2, w3, b3, w4, b4, w5, b5, out_ref):
    h = emb_ref[...]
    h = jnp.maximum(jnp.dot(h, w1[...], preferred_element_type=jnp.float32) + b1[...], 0.0)
    h = jnp.maximum(jnp.dot(h, w2[...], preferred_element_type=jnp.float32) + b2[...], 0.0)
    h = jnp.maximum(jnp.dot(h, w3[...], preferred_element_type=jnp.float32) + b3[...], 0.0)
    h = jnp.maximum(jnp.dot(h, w4[...], preferred_element_type=jnp.float32) + b4[...], 0.0)
    o = jnp.dot(h, w5[...], preferred_element_type=jnp.float32) + b5[...]
    out_ref[...] = jax.nn.sigmoid(o)


def _mlp(emb, W1, b1, W2, b2, W3, b3, W4, b4, W5, b5):
    full = lambda a: pl.BlockSpec(a.shape, lambda i: (0,) * a.ndim)
    return pl.pallas_call(
        _mlp_block,
        grid=(BATCH // BB,),
        in_specs=[
            pl.BlockSpec((BB, FIELDS * DIM), lambda i: (i, 0)),
            full(W1), full(b1), full(W2), full(b2), full(W3), full(b3),
            full(W4), full(b4), full(W5), full(b5),
        ],
        out_specs=pl.BlockSpec((BB, 1), lambda i: (i, 0)),
        out_shape=jax.ShapeDtypeStruct((BATCH, 1), jnp.float32),
    )(emb, W1, b1, W2, b2, W3, b3, W4, b4, W5, b5)


def kernel(x, table, W1, b1, W2, b2, W3, b3, W4, b4, W5, b5):
    table_lin = _relayout_table(table).reshape(16 * RELAY_BN, DIM)
    xs = (x & ~0xFFFF) | ((x & 0x1FFF) << 3) | ((x >> 13) & 7)
    idx2d = xs.reshape(IDX_ROWS, GATHER_WINDOW)
    rows = _sc_gather(table_lin, idx2d)
    emb = rows.reshape(BATCH, FIELDS * DIM)
    return _mlp(
        emb,
        W1, b1.reshape(1, -1),
        W2, b2.reshape(1, -1),
        W3, b3.reshape(1, -1),
        W4, b4.reshape(1, -1),
        W5, b5.reshape(1, -1),
    )


# speedup vs baseline: 2.3385x; 1.0518x over previous
import jax
import jax.numpy as jnp
from jax.experimental import pallas as pl
from jax.experimental.pallas import tpu as pltpu
from jax.experimental.pallas import tpu_sc as plsc

BATCH = 16384
FIELDS = 26
DIM = 16
VOCAB = 1000000
NUM_IDX = BATCH * FIELDS

GATHER_WINDOW = 128
IDX_ROWS = NUM_IDX // GATHER_WINDOW

RELAY_BN = 65536

BB = 2048


def _relay_block(tt_ref, out_ref):
    s = RELAY_BN // 8
    for c in range(8):
        out_ref[:, 16 * c:16 * (c + 1)] = tt_ref[:, c * s:(c + 1) * s].T


def _relayout_table(table):
    tt = table.T
    return pl.pallas_call(
        _relay_block,
        grid=(16,),
        in_specs=[pl.BlockSpec((DIM, RELAY_BN), lambda j: (0, j))],
        out_specs=pl.BlockSpec((RELAY_BN // 8, 128), lambda j: (j, 0)),
        out_shape=jax.ShapeDtypeStruct((16 * RELAY_BN // 8, 128), jnp.float32),
    )(tt)


def _sc_gather(table_lin, idx2d):
    mesh = plsc.VectorSubcoreMesh(core_axis_name="core", subcore_axis_name="subcore")
    n_rows = idx2d.shape[0]

    @pl.kernel(
        out_type=jax.ShapeDtypeStruct((n_rows * GATHER_WINDOW, DIM), table_lin.dtype),
        mesh=mesh,
        compiler_params=pltpu.CompilerParams(use_tc_tiling_on_sc=False),
    )
    def gather_kernel(tab_hbm, idx_hbm, out_hbm):
        def body(idx_vmem, out_vmem):
            pltpu.sync_copy(tab_hbm.at[idx_vmem.at[0]], out_vmem)

        pltpu.emit_pipeline(
            body,
            grid=(n_rows,),
            in_specs=[pl.BlockSpec((1, GATHER_WINDOW), index_map=lambda i: (i, 0))],
            out_specs=[pl.BlockSpec((GATHER_WINDOW, DIM), index_map=lambda i: (i, 0))],
            core_axis_name=("core", "subcore"),
            dimension_semantics=(pltpu.PARALLEL,),
        )(idx_hbm, out_hbm)

    return gather_kernel(table_lin, idx2d)


def _mlp_block(emb_ref, w1, b1, w2, b2, w3, b3, w4, b4, w5, b5, out_ref):
    h = emb_ref[...]
    h = jnp.maximum(jnp.dot(h, w1[...], preferred_element_type=jnp.float32) + b1[...], 0.0)
    h = jnp.maximum(jnp.dot(h, w2[...], preferred_element_type=jnp.float32) + b2[...], 0.0)
    h = jnp.maximum(jnp.dot(h, w3[...], preferred_element_type=jnp.float32) + b3[...], 0.0)
    h = jnp.maximum(jnp.dot(h, w4[...], preferred_element_type=jnp.float32) + b4[...], 0.0)
    o = jnp.dot(h, w5[...], preferred_element_type=jnp.float32) + b5[...]
    out_ref[...] = jax.nn.sigmoid(o)


def _mlp(emb, W1, b1, W2, b2, W3, b3, W4, b4, W5, b5):
    full = lambda a: pl.BlockSpec(a.shape, lambda i: (0,) * a.ndim)
    nb = emb.shape[0]
    return pl.pallas_call(
        _mlp_block,
        grid=(nb // BB,),
        in_specs=[
            pl.BlockSpec((BB, FIELDS * DIM), lambda i: (i, 0)),
            full(W1), full(b1), full(W2), full(b2), full(W3), full(b3),
            full(W4), full(b4), full(W5), full(b5),
        ],
        out_specs=pl.BlockSpec((BB, 1), lambda i: (i, 0)),
        out_shape=jax.ShapeDtypeStruct((nb, 1), jnp.float32),
    )(emb, W1, b1, W2, b2, W3, b3, W4, b4, W5, b5)


def kernel(x, table, W1, b1, W2, b2, W3, b3, W4, b4, W5, b5):
    table_lin = _relayout_table(table).reshape(16 * RELAY_BN, DIM)
    xs = (x & ~0xFFFF) | ((x & 0x1FFF) << 3) | ((x >> 13) & 7)
    biases = (b1.reshape(1, -1), b2.reshape(1, -1), b3.reshape(1, -1),
              b4.reshape(1, -1), b5.reshape(1, -1))
    outs = []
    half = BATCH // 2
    for c in range(2):
        xc = xs[c * half:(c + 1) * half]
        idx2d = xc.reshape(half * FIELDS // GATHER_WINDOW, GATHER_WINDOW)
        rows = _sc_gather(table_lin, idx2d)
        emb = rows.reshape(half, FIELDS * DIM)
        outs.append(_mlp(emb, W1, biases[0], W2, biases[1], W3, biases[2],
                         W4, biases[3], W5, biases[4]))
    return jnp.concatenate(outs, axis=0)


# speedup vs baseline: 2.3707x; 1.0138x over previous
import jax
import jax.numpy as jnp
from jax.experimental import pallas as pl
from jax.experimental.pallas import tpu as pltpu
from jax.experimental.pallas import tpu_sc as plsc

BATCH = 16384
FIELDS = 26
DIM = 16
VOCAB = 1000000
NUM_IDX = BATCH * FIELDS

GATHER_WINDOW = 128
IDX_ROWS = NUM_IDX // GATHER_WINDOW

RELAY_BN = 65536

BB = 2048


def _relay_block(tt_ref, out_ref):
    s = RELAY_BN // 8
    for c in range(8):
        out_ref[:, 16 * c:16 * (c + 1)] = tt_ref[:, c * s:(c + 1) * s].T


def _relayout_table(table):
    tt = table.T
    return pl.pallas_call(
        _relay_block,
        grid=(16,),
        in_specs=[pl.BlockSpec((DIM, RELAY_BN), lambda j: (0, j))],
        out_specs=pl.BlockSpec((RELAY_BN // 8, 128), lambda j: (j, 0)),
        out_shape=jax.ShapeDtypeStruct((16 * RELAY_BN // 8, 128), jnp.float32),
    )(tt)


def _sc_gather(table_lin, idx2d):
    mesh = plsc.VectorSubcoreMesh(core_axis_name="core", subcore_axis_name="subcore")
    n_rows = idx2d.shape[0]

    @pl.kernel(
        out_type=jax.ShapeDtypeStruct((n_rows * GATHER_WINDOW, DIM), table_lin.dtype),
        mesh=mesh,
        compiler_params=pltpu.CompilerParams(use_tc_tiling_on_sc=False),
    )
    def gather_kernel(tab_hbm, idx_hbm, out_hbm):
        def body(idx_vmem, out_vmem):
            pltpu.sync_copy(tab_hbm.at[idx_vmem.at[0]], out_vmem)

        pltpu.emit_pipeline(
            body,
            grid=(n_rows,),
            in_specs=[pl.BlockSpec((1, GATHER_WINDOW), index_map=lambda i: (i, 0))],
            out_specs=[pl.BlockSpec((GATHER_WINDOW, DIM), index_map=lambda i: (i, 0))],
            core_axis_name=("core", "subcore"),
            dimension_semantics=(pltpu.PARALLEL,),
        )(idx_hbm, out_hbm)

    return gather_kernel(table_lin, idx2d)


def _mlp_block(emb_ref, w1, b1, w2, b2, w3, b3, w4, b4, w5, b5, out_ref):
    h = emb_ref[...]
    h = jnp.maximum(jnp.dot(h, w1[...], preferred_element_type=jnp.float32) + b1[...], 0.0)
    h = jnp.maximum(jnp.dot(h, w2[...], preferred_element_type=jnp.float32) + b2[...], 0.0)
    h = jnp.maximum(jnp.dot(h, w3[...], preferred_element_type=jnp.float32) + b3[...], 0.0)
    h = jnp.maximum(jnp.dot(h, w4[...], preferred_element_type=jnp.float32) + b4[...], 0.0)
    o = jnp.dot(h, w5[...], preferred_element_type=jnp.float32) + b5[...]
    out_ref[...] = jax.nn.sigmoid(o)


def _mlp(emb, W1, b1, W2, b2, W3, b3, W4, b4, W5, b5):
    full = lambda a: pl.BlockSpec(a.shape, lambda i: (0,) * a.ndim)
    nb = emb.shape[0]
    return pl.pallas_call(
        _mlp_block,
        grid=(nb // BB,),
        in_specs=[
            pl.BlockSpec((BB, FIELDS * DIM), lambda i: (i, 0)),
            full(W1), full(b1), full(W2), full(b2), full(W3), full(b3),
            full(W4), full(b4), full(W5), full(b5),
        ],
        out_specs=pl.BlockSpec((BB, 1), lambda i: (i, 0)),
        out_shape=jax.ShapeDtypeStruct((nb, 1), jnp.float32),
    )(emb, W1, b1, W2, b2, W3, b3, W4, b4, W5, b5)


def kernel(x, table, W1, b1, W2, b2, W3, b3, W4, b4, W5, b5):
    table_lin = _relayout_table(table).reshape(16 * RELAY_BN, DIM)
    xs = (x & ~0xFFFF) | ((x & 0x1FFF) << 3) | ((x >> 13) & 7)
    biases = (b1.reshape(1, -1), b2.reshape(1, -1), b3.reshape(1, -1),
              b4.reshape(1, -1), b5.reshape(1, -1))
    outs = []
    half = BATCH // 4
    for c in range(4):
        xc = xs[c * half:(c + 1) * half]
        idx2d = xc.reshape(half * FIELDS // GATHER_WINDOW, GATHER_WINDOW)
        rows = _sc_gather(table_lin, idx2d)
        emb = rows.reshape(half, FIELDS * DIM)
        outs.append(_mlp(emb, W1, biases[0], W2, biases[1], W3, biases[2],
                         W4, biases[3], W5, biases[4]))
    return jnp.concatenate(outs, axis=0)
